# baseline (speedup 1.0000x reference)
import jax
import jax.numpy as jnp
from jax import lax
from jax.experimental import pallas as pl
from jax.experimental.pallas import tpu as pltpu
from jax.experimental.pallas import tpu_sc as plsc

R = 3
C = 4
NNZ = 5
COLS = 1024
NC = 2
NS = 16
L = 16
NW = NC * NS
W = COLS // NW


def _body(x_hbm, idx_hbm, vals_hbm, out_hbm, x_v, idx_v, vals_v, out_v):
    wid = lax.axis_index("c") * NS + lax.axis_index("s")
    base = wid * W

    pltpu.sync_copy(idx_hbm, idx_v.at[pl.ds(0, 2 * NNZ)])
    pltpu.sync_copy(vals_hbm, vals_v.at[pl.ds(0, NNZ)])
    for c in range(C):
        pltpu.sync_copy(x_hbm.at[c, pl.ds(base, W)], x_v.at[pl.ds(c * W, W)])

    lane = lax.iota(jnp.int32, L)
    valid = lane < NNZ
    idx = idx_v[...]
    rows = idx
    cols = idx.at[jnp.minimum(lane + NNZ, L - 1)].get(mode="promise_in_bounds")
    vals = vals_v[...]

    def allsum(v):
        for s in (8, 4, 2, 1):
            v = v + v.at[lane ^ s].get(mode="promise_in_bounds")
        return v

    m = [[allsum(jnp.where(valid & (rows == r) & (cols == c), vals, 0.0))
          for c in range(C)] for r in range(R)]

    for j in range(W // L):
        xs = [x_v[pl.ds(c * W + j * L, L)] for c in range(C)]
        for r in range(R):
            acc = m[r][0] * xs[0]
            for c in range(1, C):
                acc = acc + m[r][c] * xs[c]
            out_v[pl.ds(r * W + j * L, L)] = acc

    for r in range(R):
        pltpu.sync_copy(out_v.at[pl.ds(r * W, W)],
                        out_hbm.at[r, pl.ds(base, W)])


@jax.jit
def _spmm(x, idx_flat, values):
    mesh = plsc.VectorSubcoreMesh(
        core_axis_name="c", subcore_axis_name="s",
        num_cores=NC, num_subcores=NS)
    return pl.kernel(
        _body,
        out_type=jax.ShapeDtypeStruct((R, COLS), jnp.float32),
        mesh=mesh,
        scratch_types=[
            pltpu.VMEM((C * W,), jnp.float32),
            pltpu.VMEM((L,), jnp.int32),
            pltpu.VMEM((L,), jnp.float32),
            pltpu.VMEM((R * W,), jnp.float32),
        ],
    )(x, idx_flat, values)


def kernel(x, indices, values):
    return _spmm(x, indices.reshape(2 * NNZ), values)


# speedup vs baseline: 1.0329x; 1.0329x over previous
import jax
import jax.numpy as jnp
from jax import lax
from jax.experimental import pallas as pl
from jax.experimental.pallas import tpu as pltpu
from jax.experimental.pallas import tpu_sc as plsc

R = 3
C = 4
NNZ = 5
COLS = 1024
NC = 2
NS = 16
L = 16
NCHUNK = 8
W = COLS // NCHUNK
NACTIVE = R * NCHUNK


def _body(x_hbm, idx_hbm, vals_hbm, out_hbm, x_v, idx_v, vals_v, out_v, sem):
    wid = lax.axis_index("c") * NS + lax.axis_index("s")

    @pl.when(wid < NACTIVE)
    def _():
        r = wid // NCHUNK
        base = (wid % NCHUNK) * W

        cps = [
            pltpu.async_copy(idx_hbm, idx_v.at[pl.ds(0, 2 * NNZ)], sem),
            pltpu.async_copy(vals_hbm, vals_v.at[pl.ds(0, NNZ)], sem),
        ]
        for c in range(C):
            cps.append(pltpu.async_copy(
                x_hbm.at[pl.ds(c * COLS + base, W)],
                x_v.at[pl.ds(c * W, W)], sem))
        for cp in cps:
            cp.wait()

        lane = lax.iota(jnp.int32, L)
        valid = lane < NNZ
        idx = idx_v[...]
        rows = idx
        cols = idx.at[jnp.minimum(lane + NNZ, L - 1)].get(
            mode="promise_in_bounds")
        vals = vals_v[...]

        def allsum(v):
            for s in (8, 4, 2, 1):
                v = v + v.at[lane ^ s].get(mode="promise_in_bounds")
            return v

        m = [allsum(jnp.where(valid & (rows == r) & (cols == c), vals, 0.0))
             for c in range(C)]

        for j in range(W // L):
            xs = [x_v[pl.ds(c * W + j * L, L)] for c in range(C)]
            acc = m[0] * xs[0]
            for c in range(1, C):
                acc = acc + m[c] * xs[c]
            out_v[pl.ds(j * L, L)] = acc

        pltpu.sync_copy(out_v, out_hbm.at[pl.ds(r * COLS + base, W)])


@jax.jit
def _spmm(x_flat, idx_flat, values):
    mesh = plsc.VectorSubcoreMesh(
        core_axis_name="c", subcore_axis_name="s",
        num_cores=NC, num_subcores=NS)
    out_flat = pl.kernel(
        _body,
        out_type=jax.ShapeDtypeStruct((R * COLS,), jnp.float32),
        mesh=mesh,
        scratch_types=[
            pltpu.VMEM((C * W,), jnp.float32),
            pltpu.VMEM((L,), jnp.int32),
            pltpu.VMEM((L,), jnp.float32),
            pltpu.VMEM((W,), jnp.float32),
            pltpu.SemaphoreType.DMA,
        ],
    )(x_flat, idx_flat, values)
    return out_flat.reshape(R, COLS)


def kernel(x, indices, values):
    return _spmm(x.reshape(C * COLS), indices.reshape(2 * NNZ), values)


# speedup vs baseline: 1.1195x; 1.0838x over previous
import jax
import jax.numpy as jnp
from jax import lax
from jax.experimental import pallas as pl
from jax.experimental.pallas import tpu as pltpu
from jax.experimental.pallas import tpu_sc as plsc

R = 3
C = 4
NNZ = 5
COLS = 1024
NS = 16
L = 16
W = COLS // NS


def _body(x_hbm, idx_hbm, vals_hbm, out_hbm, x_v, idx_v, vals_v, out_v, sem):
    wid = lax.axis_index("s")
    base = wid * W

    cps = [
        pltpu.async_copy(idx_hbm, idx_v.at[pl.ds(0, 2 * NNZ)], sem),
        pltpu.async_copy(vals_hbm, vals_v.at[pl.ds(0, NNZ)], sem),
    ]
    for c in range(C):
        cps.append(pltpu.async_copy(
            x_hbm.at[pl.ds(c * COLS + base, W)],
            x_v.at[pl.ds(c * W, W)], sem))
    for cp in cps:
        cp.wait()

    lane = lax.iota(jnp.int32, L)
    valid = lane < NNZ
    idx = idx_v[...]
    rows = idx
    cols = idx.at[jnp.minimum(lane + NNZ, L - 1)].get(mode="promise_in_bounds")
    vals = vals_v[...]

    def allsum(v):
        for s in (8, 4, 2, 1):
            v = v + v.at[lane ^ s].get(mode="promise_in_bounds")
        return v

    m = [[allsum(jnp.where(valid & (rows == r) & (cols == c), vals, 0.0))
          for c in range(C)] for r in range(R)]

    for r in range(R):
        for j in range(W // L):
            xs = [x_v[pl.ds(c * W + j * L, L)] for c in range(C)]
            acc = m[r][0] * xs[0]
            for c in range(1, C):
                acc = acc + m[r][c] * xs[c]
            out_v[pl.ds(r * W + j * L, L)] = acc

    ocps = [pltpu.async_copy(out_v.at[pl.ds(r * W, W)],
                             out_hbm.at[pl.ds(r * COLS + base, W)], sem)
            for r in range(R)]
    for cp in ocps:
        cp.wait()


@jax.jit
def _spmm(x_flat, idx_flat, values):
    mesh = plsc.VectorSubcoreMesh(
        core_axis_name="c", subcore_axis_name="s",
        num_cores=1, num_subcores=NS)
    out_flat = pl.kernel(
        _body,
        out_type=jax.ShapeDtypeStruct((R * COLS,), jnp.float32),
        mesh=mesh,
        scratch_types=[
            pltpu.VMEM((C * W,), jnp.float32),
            pltpu.VMEM((L,), jnp.int32),
            pltpu.VMEM((L,), jnp.float32),
            pltpu.VMEM((R * W,), jnp.float32),
            pltpu.SemaphoreType.DMA,
        ],
    )(x_flat, idx_flat, values)
    return out_flat.reshape(R, COLS)


def kernel(x, indices, values):
    return _spmm(x.reshape(C * COLS), indices.reshape(2 * NNZ), values)


# speedup vs baseline: 1.1231x; 1.0032x over previous
import jax
import jax.numpy as jnp
from jax import lax
from jax.experimental import pallas as pl
from jax.experimental.pallas import tpu as pltpu
from jax.experimental.pallas import tpu_sc as plsc

R = 3
C = 4
NNZ = 5
COLS = 1024
NS = 16
L = 16
W = COLS // NS


def _body(x_hbm, idx_hbm, vals_hbm, out_hbm, x_v, idx_v, vals_v, out_v,
          sem, msem):
    wid = lax.axis_index("s")
    base = wid * W

    xcps = [pltpu.async_copy(
        x_hbm.at[pl.ds(c * COLS + base, W)],
        x_v.at[pl.ds(c * W, W)], sem) for c in range(C)]
    mcps = [
        pltpu.async_copy(idx_hbm, idx_v.at[pl.ds(0, 2 * NNZ)], msem),
        pltpu.async_copy(vals_hbm, vals_v.at[pl.ds(0, NNZ)], msem),
    ]
    for cp in mcps:
        cp.wait()

    lane = lax.iota(jnp.int32, L)
    valid = lane < NNZ
    idx = idx_v[...]
    rows = idx
    cols = idx.at[jnp.minimum(lane + NNZ, L - 1)].get(mode="promise_in_bounds")
    vals = vals_v[...]

    def allsum(v):
        for s in (8, 4, 2, 1):
            v = v + v.at[lane ^ s].get(mode="promise_in_bounds")
        return v

    m = [[allsum(jnp.where(valid & (rows == r) & (cols == c), vals, 0.0))
          for c in range(C)] for r in range(R)]

    for cp in xcps:
        cp.wait()

    ocps = []
    for r in range(R):
        for j in range(W // L):
            xs = [x_v[pl.ds(c * W + j * L, L)] for c in range(C)]
            acc = m[r][0] * xs[0]
            for c in range(1, C):
                acc = acc + m[r][c] * xs[c]
            out_v[pl.ds(r * W + j * L, L)] = acc
        ocps.append(pltpu.async_copy(
            out_v.at[pl.ds(r * W, W)],
            out_hbm.at[pl.ds(r * COLS + base, W)], sem))
    for cp in ocps:
        cp.wait()


@jax.jit
def _spmm(x_flat, idx_flat, values):
    mesh = plsc.VectorSubcoreMesh(
        core_axis_name="c", subcore_axis_name="s",
        num_cores=1, num_subcores=NS)
    out_flat = pl.kernel(
        _body,
        out_type=jax.ShapeDtypeStruct((R * COLS,), jnp.float32),
        mesh=mesh,
        scratch_types=[
            pltpu.VMEM((C * W,), jnp.float32),
            pltpu.VMEM((L,), jnp.int32),
            pltpu.VMEM((L,), jnp.float32),
            pltpu.VMEM((R * W,), jnp.float32),
            pltpu.SemaphoreType.DMA,
            pltpu.SemaphoreType.DMA,
        ],
    )(x_flat, idx_flat, values)
    return out_flat.reshape(R, COLS)


def kernel(x, indices, values):
    return _spmm(x.reshape(C * COLS), indices.reshape(2 * NNZ), values)
